# baseline (speedup 1.0000x reference)
import functools

import jax
import jax.numpy as jnp
from jax.experimental import pallas as pl

_CORE_DIM = 256
_N = 1024
_TEMP = 5.0
_BLK = 2048


def _body(z_ref, p_ref, acts_ref, assign_ref, util_ref):
    i = pl.program_id(0)

    z = z_ref[...]
    p = p_ref[...]
    cos = jax.lax.dot_general(
        z, p, (((1,), (1,)), ((), ())),
        preferred_element_type=jnp.float32,
    )

    v = cos
    neg = jnp.float32(-jnp.inf)
    lanes = 128
    colb = jax.lax.broadcasted_iota(
        jnp.int32, (v.shape[0], lanes), 1).astype(jnp.float32)
    r1 = v[:, :lanes]
    r2 = jnp.full_like(r1, neg)
    ii = colb
    for k in range(1, v.shape[1] // lanes):
        vk = v[:, k * lanes:(k + 1) * lanes]
        ii = jnp.where(vk > r1, colb + float(k * lanes), ii)
        lose1 = jnp.minimum(r1, vk)
        r1 = jnp.maximum(r1, vk)
        r2 = jnp.maximum(r2, lose1)
    m0 = jnp.max(r1, axis=-1, keepdims=True)
    i0 = jnp.min(jnp.where(r1 == m0, ii, float(_N)), axis=-1, keepdims=True)
    hl = r1 == m0
    m1 = jnp.max(jnp.where(hl, r2, r1), axis=-1, keepdims=True)
    m2 = jnp.max(jnp.where(hl, jnp.where(r2 == m1, neg, r2),
                           jnp.where(r1 == m1, r2, r1)),
                 axis=-1, keepdims=True)

    e1 = jnp.exp((m1 - m0) / _TEMP)
    e2 = jnp.exp((m2 - m0) / _TEMP)
    s = 1.0 + e1 + e2
    w0 = 1.0 / s
    w1 = e1 / s
    w2 = e2 / s

    acts = jnp.where(v == m0, w0,
                     jnp.where(v == m1, w1,
                               jnp.where(v == m2, w2, 0.0)))
    acts_ref[...] = acts

    assign_ref[...] = i0.astype(jnp.int32)

    @pl.when(i == 0)
    def _():
        util_ref[...] = jnp.zeros_like(util_ref)

    util_ref[...] += jnp.sum(
        jnp.where(v >= m0, 1.0, 0.0), axis=0, keepdims=True)


@functools.partial(jax.jit, static_argnames=())
def kernel(z, prototypes):
    b, d = z.shape
    n = prototypes.shape[0]
    nb = b // _BLK

    acts, assign2d, util = pl.pallas_call(
        _body,
        grid=(nb,),
        in_specs=[
            pl.BlockSpec((_BLK, d), lambda i: (i, 0)),
            pl.BlockSpec((n, d), lambda i: (0, 0)),
        ],
        out_specs=[
            pl.BlockSpec((_BLK, n), lambda i: (i, 0)),
            pl.BlockSpec((_BLK, 1), lambda i: (i, 0)),
            pl.BlockSpec((1, n), lambda i: (0, 0)),
        ],
        out_shape=[
            jax.ShapeDtypeStruct((b, n), jnp.float32),
            jax.ShapeDtypeStruct((b, 1), jnp.int32),
            jax.ShapeDtypeStruct((1, n), jnp.float32),
        ],
    )(z, prototypes)

    return acts, assign2d.reshape(b), util.reshape(n)


# speedup vs baseline: 1.0331x; 1.0331x over previous
import functools

import jax
import jax.numpy as jnp
from jax.experimental import pallas as pl

_CORE_DIM = 256
_N = 1024
_TEMP = 5.0
_BLK = 2048


def _body(z_ref, p_ref, acts_ref, assign_ref, util_ref):
    i = pl.program_id(0)

    z = z_ref[...]
    p = p_ref[...]
    cos = jax.lax.dot_general(
        z, p, (((1,), (1,)), ((), ())),
        preferred_element_type=jnp.float32,
    )

    v = cos
    neg = jnp.float32(-jnp.inf)
    lanes = 128
    colb = jax.lax.broadcasted_iota(
        jnp.int32, (v.shape[0], lanes), 1).astype(jnp.float32)
    r1 = v[:, :lanes]
    r2 = jnp.full_like(r1, neg)
    ii = colb
    for k in range(1, v.shape[1] // lanes):
        vk = v[:, k * lanes:(k + 1) * lanes]
        ii = jnp.where(vk > r1, colb + float(k * lanes), ii)
        lose1 = jnp.minimum(r1, vk)
        r1 = jnp.maximum(r1, vk)
        r2 = jnp.maximum(r2, lose1)
    m0 = jnp.max(r1, axis=-1, keepdims=True)
    i0 = jnp.min(jnp.where(r1 == m0, ii, float(_N)), axis=-1, keepdims=True)
    hl = r1 == m0
    m1 = jnp.max(jnp.where(hl, r2, r1), axis=-1, keepdims=True)
    m2 = jnp.max(jnp.where(hl, jnp.where(r2 == m1, neg, r2),
                           jnp.where(r1 == m1, r2, r1)),
                 axis=-1, keepdims=True)

    e1 = jnp.exp((m1 - m0) / _TEMP)
    e2 = jnp.exp((m2 - m0) / _TEMP)
    s = 1.0 + e1 + e2
    w0 = 1.0 / s
    w1 = e1 / s
    w2 = e2 / s

    acts = jnp.where(v == m0, w0,
                     jnp.where(v == m1, w1,
                               jnp.where(v == m2, w2, 0.0)))
    acts_ref[...] = acts

    assign_ref[...] = i0.astype(jnp.int32)

    @pl.when(i == 0)
    def _():
        util_ref[...] = jnp.zeros_like(util_ref)

    maskf = jnp.where(v >= m0, 1.0, 0.0)
    ones = jnp.ones((1, v.shape[0]), jnp.float32)
    util_ref[...] += jax.lax.dot_general(
        ones, maskf, (((1,), (0,)), ((), ())),
        preferred_element_type=jnp.float32)


@functools.partial(jax.jit, static_argnames=())
def kernel(z, prototypes):
    b, d = z.shape
    n = prototypes.shape[0]
    nb = b // _BLK

    acts, assign2d, util = pl.pallas_call(
        _body,
        grid=(nb,),
        in_specs=[
            pl.BlockSpec((_BLK, d), lambda i: (i, 0)),
            pl.BlockSpec((n, d), lambda i: (0, 0)),
        ],
        out_specs=[
            pl.BlockSpec((_BLK, n), lambda i: (i, 0)),
            pl.BlockSpec((_BLK, 1), lambda i: (i, 0)),
            pl.BlockSpec((1, n), lambda i: (0, 0)),
        ],
        out_shape=[
            jax.ShapeDtypeStruct((b, n), jnp.float32),
            jax.ShapeDtypeStruct((b, 1), jnp.int32),
            jax.ShapeDtypeStruct((1, n), jnp.float32),
        ],
    )(z, prototypes)

    return acts, assign2d.reshape(b), util.reshape(n)
